# baseline (speedup 1.0000x reference)
import functools

import jax
import jax.numpy as jnp
from jax import lax
from jax.experimental import pallas as pl
from jax.experimental.pallas import tpu as pltpu
from jax.experimental.pallas import tpu_sc as plsc

B = 4096
F_CAT = 26
V_CAT = 100000
F_DENSE = 13
F_CROSS = 6
V_CROSS = 1000000

NC = 2
NS = 16
L = 16
NW = NC * NS
BPW = B // NW
NCH = BPW // L


def _splat(aux_vec, i):
    return jnp.full((L,), aux_vec[i], jnp.float32)


def _body(catT, crossT, denseT, wcat, wcross, aux, out,
          idx_cat_v, idx_cross_v, vals_cat_v, vals_cross_v,
          dense_v, acc_v, aux_v, sem):
    wid = lax.axis_index("s") * NC + lax.axis_index("c")
    base = wid * BPW

    pltpu.sync_copy(catT.at[:, pl.ds(base, BPW)], idx_cat_v)
    pltpu.sync_copy(crossT.at[:, pl.ds(base, BPW)], idx_cross_v)
    pltpu.sync_copy(denseT.at[:, pl.ds(base, BPW)], dense_v)
    pltpu.sync_copy(aux, aux_v)

    for f in range(1, F_CAT):
        for c in range(NCH):
            sl = pl.ds(c * L, L)
            idx_cat_v[f, sl] = idx_cat_v[f, sl] + jnp.int32(f * V_CAT)
    for f in range(1, F_CROSS):
        for c in range(NCH):
            sl = pl.ds(c * L, L)
            idx_cross_v[f, sl] = idx_cross_v[f, sl] + jnp.int32(f * V_CROSS)

    copies = []
    for f in range(F_CAT):
        copies.append(
            pltpu.async_copy(wcat.at[idx_cat_v.at[f]], vals_cat_v.at[f], sem))
    for f in range(F_CROSS):
        copies.append(
            pltpu.async_copy(wcross.at[idx_cross_v.at[f]], vals_cross_v.at[f],
                             sem))
    for cp in copies:
        cp.wait()

    aux_vec = aux_v[:]
    wsplats = [_splat(aux_vec, f) for f in range(F_DENSE)]
    bias_splat = _splat(aux_vec, F_DENSE)
    for c in range(NCH):
        sl = pl.ds(c * L, L)
        s = vals_cat_v[0, sl]
        for f in range(1, F_CAT):
            s = s + vals_cat_v[f, sl]
        for f in range(F_CROSS):
            s = s + vals_cross_v[f, sl]
        for f in range(F_DENSE):
            s = s + dense_v[f, sl] * wsplats[f]
        acc_v[sl] = s + bias_splat

    pltpu.sync_copy(acc_v, out.at[pl.ds(base, BPW)])


@jax.jit
def _poly2(catT, crossT, denseT, wcat, wcross, aux):
    mesh = plsc.VectorSubcoreMesh(core_axis_name="c", subcore_axis_name="s")
    return pl.kernel(
        _body,
        out_type=jax.ShapeDtypeStruct((B,), jnp.float32),
        mesh=mesh,
        scratch_types=[
            pltpu.VMEM((F_CAT, BPW), jnp.int32),
            pltpu.VMEM((F_CROSS, BPW), jnp.int32),
            pltpu.VMEM((F_CAT, BPW), jnp.float32),
            pltpu.VMEM((F_CROSS, BPW), jnp.float32),
            pltpu.VMEM((F_DENSE, BPW), jnp.float32),
            pltpu.VMEM((BPW,), jnp.float32),
            pltpu.VMEM((L,), jnp.float32),
            pltpu.SemaphoreType.DMA,
        ],
    )(catT, crossT, denseT, wcat, wcross, aux)


def kernel(cat_idx, dense_x, cross_idx, W_cat, W_dense, W_cross, bias):
    catT = cat_idx.astype(jnp.int32).T
    crossT = cross_idx.astype(jnp.int32).T
    denseT = dense_x.T
    wcat = W_cat.reshape(-1)
    wcross = W_cross.reshape(-1)
    aux = jnp.concatenate(
        [W_dense.reshape(-1), bias.reshape(-1),
         jnp.zeros((L - F_DENSE - 1,), jnp.float32)])
    out = _poly2(catT, crossT, denseT, wcat, wcross, aux)
    return out.reshape(B, 1)


# speedup vs baseline: 3.5862x; 3.5862x over previous
import functools

import jax
import jax.numpy as jnp
from jax import lax
from jax.experimental import pallas as pl
from jax.experimental.pallas import tpu as pltpu
from jax.experimental.pallas import tpu_sc as plsc

B = 4096
F_CAT = 26
V_CAT = 100000
F_DENSE = 13
F_CROSS = 6
V_CROSS = 1000000

NC = 2
NS = 16
L = 16
NW = NC * NS
BPW = B // NW
NCH = BPW // L


def _splat(aux_vec, i):
    return jnp.full((L,), aux_vec[i], jnp.float32)


def _body(catT, crossT, denseT, wcat, wcross_a, wcross_b, aux, out,
          idx_cat_v, idx_cross_v, vals_cat_v, vals_cross_v,
          dense_v, acc_v, aux_v, sem):
    wid = lax.axis_index("s") * NC + lax.axis_index("c")
    base = wid * BPW

    pltpu.sync_copy(catT.at[:, pl.ds(base, BPW)], idx_cat_v)
    pltpu.sync_copy(crossT.at[:, pl.ds(base, BPW)], idx_cross_v)
    pltpu.sync_copy(denseT.at[:, pl.ds(base, BPW)], dense_v)
    pltpu.sync_copy(aux, aux_v)

    for f in range(1, F_CAT):
        for c in range(NCH):
            sl = pl.ds(c * L, L)
            idx_cat_v[f, sl] = idx_cat_v[f, sl] + jnp.int32(f * V_CAT)
    for f in range(1, F_CROSS):
        f_local = f % (F_CROSS // 2)
        if f_local == 0:
            continue
        for c in range(NCH):
            sl = pl.ds(c * L, L)
            idx_cross_v[f, sl] = idx_cross_v[f, sl] + jnp.int32(
                f_local * V_CROSS)

    copies = []
    for f in range(F_CAT):
        copies.append(
            pltpu.async_copy(wcat.at[idx_cat_v.at[f]], vals_cat_v.at[f], sem))
    for f in range(F_CROSS):
        half = wcross_a if f < F_CROSS // 2 else wcross_b
        copies.append(
            pltpu.async_copy(half.at[idx_cross_v.at[f]], vals_cross_v.at[f],
                             sem))
    for cp in copies:
        cp.wait()

    aux_vec = aux_v[:]
    wsplats = [_splat(aux_vec, f) for f in range(F_DENSE)]
    bias_splat = _splat(aux_vec, F_DENSE)
    for c in range(NCH):
        sl = pl.ds(c * L, L)
        s = vals_cat_v[0, sl]
        for f in range(1, F_CAT):
            s = s + vals_cat_v[f, sl]
        for f in range(F_CROSS):
            s = s + vals_cross_v[f, sl]
        for f in range(F_DENSE):
            s = s + dense_v[f, sl] * wsplats[f]
        acc_v[sl] = s + bias_splat

    pltpu.sync_copy(acc_v, out.at[pl.ds(base, BPW)])


@jax.jit
def _poly2(catT, crossT, denseT, wcat, wcross_a, wcross_b, aux):
    mesh = plsc.VectorSubcoreMesh(core_axis_name="c", subcore_axis_name="s")
    return pl.kernel(
        _body,
        out_type=jax.ShapeDtypeStruct((B,), jnp.float32),
        mesh=mesh,
        scratch_types=[
            pltpu.VMEM((F_CAT, BPW), jnp.int32),
            pltpu.VMEM((F_CROSS, BPW), jnp.int32),
            pltpu.VMEM((F_CAT, BPW), jnp.float32),
            pltpu.VMEM((F_CROSS, BPW), jnp.float32),
            pltpu.VMEM((F_DENSE, BPW), jnp.float32),
            pltpu.VMEM((BPW,), jnp.float32),
            pltpu.VMEM((L,), jnp.float32),
            pltpu.SemaphoreType.DMA,
        ],
    )(catT, crossT, denseT, wcat, wcross_a, wcross_b, aux)


def kernel(cat_idx, dense_x, cross_idx, W_cat, W_dense, W_cross, bias):
    catT = cat_idx.astype(jnp.int32).T
    crossT = cross_idx.astype(jnp.int32).T
    denseT = dense_x.T
    wcat = W_cat.reshape(-1)
    h = F_CROSS // 2
    wcross_a = W_cross[:h].reshape(-1)
    wcross_b = W_cross[h:].reshape(-1)
    aux = jnp.concatenate(
        [W_dense.reshape(-1), bias.reshape(-1),
         jnp.zeros((L - F_DENSE - 1,), jnp.float32)])
    out = _poly2(catT, crossT, denseT, wcat, wcross_a, wcross_b, aux)
    return out.reshape(B, 1)


# speedup vs baseline: 7.4850x; 2.0872x over previous
import jax
import jax.numpy as jnp
from jax import lax
from jax.experimental import pallas as pl
from jax.experimental.pallas import tpu as pltpu
from jax.experimental.pallas import tpu_sc as plsc

B = 4096
F_CAT = 26
V_CAT = 100000
F_DENSE = 13
F_CROSS = 6
V_CROSS = 1000000

NC = 2
NS = 16
L = 16
NW = NC * NS
BPW = B // NW
NCH = BPW // L
CW = 8


def _splat(aux_vec, i):
    return jnp.full((L,), aux_vec[i], jnp.float32)


def _body(catT, crossT, denseT, wcat, wcross, aux, out,
          idx_cat_v, idx_cross_v, vals_cat_v, cross8_v,
          dense_v, acc_v, aux_v, sem, csem):
    wid = lax.axis_index("s") * NC + lax.axis_index("c")
    base = wid * BPW

    pltpu.sync_copy(catT.at[:, pl.ds(base, BPW)], idx_cat_v)
    pltpu.sync_copy(crossT.at[:, pl.ds(base, BPW)], idx_cross_v)
    pltpu.sync_copy(denseT.at[:, pl.ds(base, BPW)], dense_v)
    pltpu.sync_copy(aux, aux_v)

    for f in range(F_CROSS):
        def cross_chunk(c, carry, f=f):
            ivec = idx_cross_v[f, pl.ds(c * L, L)]
            col8 = ivec & jnp.int32(~(CW - 1))
            for l in range(L):
                off = pl.multiple_of(col8[l], CW)
                pltpu.async_copy(
                    wcross.at[f, pl.ds(off, CW)],
                    cross8_v.at[f, pl.ds(c * (L * CW) + l * CW, CW)],
                    csem)
            pltpu.make_async_copy(
                wcross.at[f, pl.ds(0, L * CW)],
                cross8_v.at[f, pl.ds(c * (L * CW), L * CW)],
                csem).wait()
            return carry
        lax.fori_loop(0, NCH, cross_chunk, 0)

    for f in range(1, F_CAT):
        for c in range(NCH):
            sl = pl.ds(c * L, L)
            idx_cat_v[f, sl] = idx_cat_v[f, sl] + jnp.int32(f * V_CAT)
    copies = []
    for f in range(F_CAT):
        copies.append(
            pltpu.async_copy(wcat.at[idx_cat_v.at[f]], vals_cat_v.at[f], sem))
    for cp in copies:
        cp.wait()

    aux_vec = aux_v[:]
    wsplats = [_splat(aux_vec, f) for f in range(F_DENSE)]
    bias_splat = _splat(aux_vec, F_DENSE)
    lanes8 = lax.iota(jnp.int32, L) * jnp.int32(CW)
    for c in range(NCH):
        sl = pl.ds(c * L, L)
        s = vals_cat_v[0, sl]
        for f in range(1, F_CAT):
            s = s + vals_cat_v[f, sl]
        for f in range(F_CROSS):
            gidx = (jnp.int32(c * (L * CW)) + lanes8
                    + (idx_cross_v[f, sl] & jnp.int32(CW - 1)))
            s = s + plsc.load_gather(cross8_v,
                                     [jnp.full((L,), f, jnp.int32), gidx])
        for f in range(F_DENSE):
            s = s + dense_v[f, sl] * wsplats[f]
        acc_v[sl] = s + bias_splat

    pltpu.sync_copy(acc_v, out.at[pl.ds(base, BPW)])


@jax.jit
def _poly2(catT, crossT, denseT, wcat, wcross, aux):
    mesh = plsc.VectorSubcoreMesh(core_axis_name="c", subcore_axis_name="s")
    return pl.kernel(
        _body,
        out_type=jax.ShapeDtypeStruct((B,), jnp.float32),
        mesh=mesh,
        scratch_types=[
            pltpu.VMEM((F_CAT, BPW), jnp.int32),
            pltpu.VMEM((F_CROSS, BPW), jnp.int32),
            pltpu.VMEM((F_CAT, BPW), jnp.float32),
            pltpu.VMEM((F_CROSS, BPW * CW), jnp.float32),
            pltpu.VMEM((F_DENSE, BPW), jnp.float32),
            pltpu.VMEM((BPW,), jnp.float32),
            pltpu.VMEM((L,), jnp.float32),
            pltpu.SemaphoreType.DMA,
            pltpu.SemaphoreType.DMA,
        ],
        compiler_params=pltpu.CompilerParams(needs_layout_passes=False),
    )(catT, crossT, denseT, wcat, wcross, aux)


def kernel(cat_idx, dense_x, cross_idx, W_cat, W_dense, W_cross, bias):
    catT = cat_idx.astype(jnp.int32).T
    crossT = cross_idx.astype(jnp.int32).T
    denseT = dense_x.T
    wcat = W_cat.reshape(-1)
    aux = jnp.concatenate(
        [W_dense.reshape(-1), bias.reshape(-1),
         jnp.zeros((L - F_DENSE - 1,), jnp.float32)])
    out = _poly2(catT, crossT, denseT, wcat, W_cross, aux)
    return out.reshape(B, 1)


# speedup vs baseline: 9.2083x; 1.2302x over previous
import jax
import jax.numpy as jnp
from jax import lax
from jax.experimental import pallas as pl
from jax.experimental.pallas import tpu as pltpu
from jax.experimental.pallas import tpu_sc as plsc

B = 4096
F_CAT = 26
V_CAT = 100000
F_DENSE = 13
F_CROSS = 6
V_CROSS = 1000000

NC = 2
NS = 16
L = 16
NW = NC * NS
BPW = B // NW
NCH = BPW // L
CW = 8


def _splat(aux_vec, i):
    return jnp.full((L,), aux_vec[i], jnp.float32)


def _body(catT, crossT, denseT, wcat, wcross, aux, out,
          idx_cat_v, idx_cross_v, vals_cat_v, cross8_v,
          dense_v, acc_v, aux_v, sem, csem):
    wid = lax.axis_index("s") * NC + lax.axis_index("c")
    base = wid * BPW

    pltpu.sync_copy(catT.at[:, pl.ds(base, BPW)], idx_cat_v)
    pltpu.sync_copy(crossT.at[:, pl.ds(base, BPW)], idx_cross_v)
    pltpu.sync_copy(denseT.at[:, pl.ds(base, BPW)], dense_v)
    pltpu.sync_copy(aux, aux_v)

    for f in range(1, F_CAT):
        for c in range(NCH):
            sl = pl.ds(c * L, L)
            idx_cat_v[f, sl] = idx_cat_v[f, sl] + jnp.int32(f * V_CAT)
    copies = []
    for f in range(F_CAT):
        copies.append(
            pltpu.async_copy(wcat.at[idx_cat_v.at[f]], vals_cat_v.at[f], sem))

    for f in range(F_CROSS):
        def cross_chunk(c, carry, f=f):
            ivec = idx_cross_v[f, pl.ds(c * L, L)]
            col8 = ivec & jnp.int32(~(CW - 1))
            for l in range(L):
                off = pl.multiple_of(col8[l], CW)
                pltpu.async_copy(
                    wcross.at[f, pl.ds(off, CW)],
                    cross8_v.at[f, pl.ds(c * (L * CW) + l * CW, CW)],
                    csem)

            @pl.when(c > 0)
            def _drain_prev():
                pltpu.make_async_copy(
                    wcross.at[f, pl.ds(0, L * CW)],
                    cross8_v.at[f, pl.ds((c - 1) * (L * CW), L * CW)],
                    csem).wait()
            return carry
        lax.fori_loop(0, NCH, cross_chunk, 0)
        pltpu.make_async_copy(
            wcross.at[f, pl.ds(0, L * CW)],
            cross8_v.at[f, pl.ds((NCH - 1) * (L * CW), L * CW)],
            csem).wait()

    for cp in copies:
        cp.wait()

    aux_vec = aux_v[:]
    wsplats = [_splat(aux_vec, f) for f in range(F_DENSE)]
    bias_splat = _splat(aux_vec, F_DENSE)
    lanes8 = lax.iota(jnp.int32, L) * jnp.int32(CW)
    for c in range(NCH):
        sl = pl.ds(c * L, L)
        s = vals_cat_v[0, sl]
        for f in range(1, F_CAT):
            s = s + vals_cat_v[f, sl]
        for f in range(F_CROSS):
            gidx = (jnp.int32(c * (L * CW)) + lanes8
                    + (idx_cross_v[f, sl] & jnp.int32(CW - 1)))
            s = s + plsc.load_gather(cross8_v,
                                     [jnp.full((L,), f, jnp.int32), gidx])
        for f in range(F_DENSE):
            s = s + dense_v[f, sl] * wsplats[f]
        acc_v[sl] = s + bias_splat

    pltpu.sync_copy(acc_v, out.at[pl.ds(base, BPW)])


@jax.jit
def _poly2(catT, crossT, denseT, wcat, wcross, aux):
    mesh = plsc.VectorSubcoreMesh(core_axis_name="c", subcore_axis_name="s")
    return pl.kernel(
        _body,
        out_type=jax.ShapeDtypeStruct((B,), jnp.float32),
        mesh=mesh,
        scratch_types=[
            pltpu.VMEM((F_CAT, BPW), jnp.int32),
            pltpu.VMEM((F_CROSS, BPW), jnp.int32),
            pltpu.VMEM((F_CAT, BPW), jnp.float32),
            pltpu.VMEM((F_CROSS, BPW * CW), jnp.float32),
            pltpu.VMEM((F_DENSE, BPW), jnp.float32),
            pltpu.VMEM((BPW,), jnp.float32),
            pltpu.VMEM((L,), jnp.float32),
            pltpu.SemaphoreType.DMA,
            pltpu.SemaphoreType.DMA,
        ],
        compiler_params=pltpu.CompilerParams(needs_layout_passes=False),
    )(catT, crossT, denseT, wcat, wcross, aux)


def kernel(cat_idx, dense_x, cross_idx, W_cat, W_dense, W_cross, bias):
    catT = cat_idx.astype(jnp.int32).T
    crossT = cross_idx.astype(jnp.int32).T
    denseT = dense_x.T
    wcat = W_cat.reshape(-1)
    aux = jnp.concatenate(
        [W_dense.reshape(-1), bias.reshape(-1),
         jnp.zeros((L - F_DENSE - 1,), jnp.float32)])
    out = _poly2(catT, crossT, denseT, wcat, W_cross, aux)
    return out.reshape(B, 1)


# speedup vs baseline: 9.7298x; 1.0566x over previous
import jax
import jax.numpy as jnp
from jax import lax
from jax.experimental import pallas as pl
from jax.experimental.pallas import tpu as pltpu
from jax.experimental.pallas import tpu_sc as plsc

B = 4096
F_CAT = 26
V_CAT = 100000
F_DENSE = 13
F_CROSS = 6
V_CROSS = 1000000

NC = 2
NS = 16
L = 16
NW = NC * NS
BPW = B // NW
NCH = BPW // L
CW = 8


def _splat(aux_vec, i):
    return jnp.full((L,), aux_vec[i], jnp.float32)


def _body(catT, crossT, denseT, wcat, wcross, aux, out,
          idx_cat_v, idx_cross_v, vals_cat_v, cross8_v,
          dense_v, acc_v, aux_v, sem, csem):
    wid = lax.axis_index("s") * NC + lax.axis_index("c")
    base = wid * BPW

    pltpu.sync_copy(catT.at[:, pl.ds(base, BPW)], idx_cat_v)
    pltpu.sync_copy(crossT.at[:, pl.ds(base, BPW)], idx_cross_v)
    pltpu.sync_copy(denseT.at[:, pl.ds(base, BPW)], dense_v)
    pltpu.sync_copy(aux, aux_v)

    for f in range(1, F_CAT):
        for c in range(NCH):
            sl = pl.ds(c * L, L)
            idx_cat_v[f, sl] = idx_cat_v[f, sl] + jnp.int32(f * V_CAT)
    copies = []
    for f in range(F_CAT):
        copies.append(
            pltpu.async_copy(wcat.at[idx_cat_v.at[f]], vals_cat_v.at[f], sem))

    for f in range(F_CROSS):
        def cross_chunk(c, carry, f=f):
            ivec = idx_cross_v[f, pl.ds(c * L, L)]
            col8 = ivec & jnp.int32(~(CW - 1))
            for l in range(L):
                off = pl.multiple_of(col8[l], CW)
                pltpu.async_copy(
                    wcross.at[f, pl.ds(off, CW)],
                    cross8_v.at[f, pl.ds(c * (L * CW) + l * CW, CW)],
                    csem)

            @pl.when(c > 1)
            def _drain_prev():
                pltpu.make_async_copy(
                    wcross.at[f, pl.ds(0, L * CW)],
                    cross8_v.at[f, pl.ds((c - 2) * (L * CW), L * CW)],
                    csem).wait()
            return carry
        lax.fori_loop(0, NCH, cross_chunk, 0)
        pltpu.make_async_copy(
            wcross.at[f, pl.ds(0, 2 * L * CW)],
            cross8_v.at[f, pl.ds((NCH - 2) * (L * CW), 2 * L * CW)],
            csem).wait()

    for cp in copies:
        cp.wait()

    aux_vec = aux_v[:]
    wsplats = [_splat(aux_vec, f) for f in range(F_DENSE)]
    bias_splat = _splat(aux_vec, F_DENSE)
    lanes8 = lax.iota(jnp.int32, L) * jnp.int32(CW)
    for c in range(NCH):
        sl = pl.ds(c * L, L)
        s = vals_cat_v[0, sl]
        for f in range(1, F_CAT):
            s = s + vals_cat_v[f, sl]
        for f in range(F_CROSS):
            gidx = (jnp.int32(c * (L * CW)) + lanes8
                    + (idx_cross_v[f, sl] & jnp.int32(CW - 1)))
            s = s + plsc.load_gather(cross8_v,
                                     [jnp.full((L,), f, jnp.int32), gidx])
        for f in range(F_DENSE):
            s = s + dense_v[f, sl] * wsplats[f]
        acc_v[sl] = s + bias_splat

    pltpu.sync_copy(acc_v, out.at[pl.ds(base, BPW)])


@jax.jit
def _poly2(catT, crossT, denseT, wcat, wcross, aux):
    mesh = plsc.VectorSubcoreMesh(core_axis_name="c", subcore_axis_name="s")
    return pl.kernel(
        _body,
        out_type=jax.ShapeDtypeStruct((B,), jnp.float32),
        mesh=mesh,
        scratch_types=[
            pltpu.VMEM((F_CAT, BPW), jnp.int32),
            pltpu.VMEM((F_CROSS, BPW), jnp.int32),
            pltpu.VMEM((F_CAT, BPW), jnp.float32),
            pltpu.VMEM((F_CROSS, BPW * CW), jnp.float32),
            pltpu.VMEM((F_DENSE, BPW), jnp.float32),
            pltpu.VMEM((BPW,), jnp.float32),
            pltpu.VMEM((L,), jnp.float32),
            pltpu.SemaphoreType.DMA,
            pltpu.SemaphoreType.DMA,
        ],
        compiler_params=pltpu.CompilerParams(needs_layout_passes=False),
    )(catT, crossT, denseT, wcat, wcross, aux)


def kernel(cat_idx, dense_x, cross_idx, W_cat, W_dense, W_cross, bias):
    catT = cat_idx.astype(jnp.int32).T
    crossT = cross_idx.astype(jnp.int32).T
    denseT = dense_x.T
    wcat = W_cat.reshape(-1)
    aux = jnp.concatenate(
        [W_dense.reshape(-1), bias.reshape(-1),
         jnp.zeros((L - F_DENSE - 1,), jnp.float32)])
    out = _poly2(catT, crossT, denseT, wcat, W_cross, aux)
    return out.reshape(B, 1)


# speedup vs baseline: 10.0173x; 1.0296x over previous
import jax
import jax.numpy as jnp
from jax import lax
from jax.experimental import pallas as pl
from jax.experimental.pallas import tpu as pltpu
from jax.experimental.pallas import tpu_sc as plsc

B = 4096
F_CAT = 26
V_CAT = 100000
F_DENSE = 13
F_CROSS = 6
V_CROSS = 1000000

NC = 2
NS = 16
L = 16
NW = NC * NS
BPW = B // NW
NCH = BPW // L
CW = 8


def _splat(aux_vec, i):
    return jnp.full((L,), aux_vec[i], jnp.float32)


def _body(catT, crossT, denseT, wcat, wcross, aux, out,
          idx_cat_v, idx_cross_v, vals_cat_v, cross8_v,
          dense_v, acc_v, aux_v, sem, csem):
    wid = lax.axis_index("s") * NC + lax.axis_index("c")
    base = wid * BPW

    stage = [
        pltpu.async_copy(catT.at[:, pl.ds(base, BPW)], idx_cat_v, sem),
        pltpu.async_copy(crossT.at[:, pl.ds(base, BPW)], idx_cross_v, sem),
        pltpu.async_copy(denseT.at[:, pl.ds(base, BPW)], dense_v, sem),
        pltpu.async_copy(aux, aux_v, sem),
    ]
    for cp in stage:
        cp.wait()

    for f in range(1, F_CAT):
        for c in range(NCH):
            sl = pl.ds(c * L, L)
            idx_cat_v[f, sl] = idx_cat_v[f, sl] + jnp.int32(f * V_CAT)
    copies = []
    for f in range(F_CAT):
        copies.append(
            pltpu.async_copy(wcat.at[idx_cat_v.at[f]], vals_cat_v.at[f], sem))

    for f in range(F_CROSS):
        def cross_chunk(c, carry, f=f):
            ivec = idx_cross_v[f, pl.ds(c * L, L)]
            col8 = ivec & jnp.int32(~(CW - 1))
            for l in range(L):
                off = pl.multiple_of(col8[l], CW)
                pltpu.async_copy(
                    wcross.at[f, pl.ds(off, CW)],
                    cross8_v.at[f, pl.ds(c * (L * CW) + l * CW, CW)],
                    csem)

            @pl.when(c > 1)
            def _drain_prev():
                pltpu.make_async_copy(
                    wcross.at[f, pl.ds(0, L * CW)],
                    cross8_v.at[f, pl.ds((c - 2) * (L * CW), L * CW)],
                    csem).wait()
            return carry
        lax.fori_loop(0, NCH, cross_chunk, 0)
        pltpu.make_async_copy(
            wcross.at[f, pl.ds(0, 2 * L * CW)],
            cross8_v.at[f, pl.ds((NCH - 2) * (L * CW), 2 * L * CW)],
            csem).wait()

    for cp in copies:
        cp.wait()

    aux_vec = aux_v[:]
    wsplats = [_splat(aux_vec, f) for f in range(F_DENSE)]
    bias_splat = _splat(aux_vec, F_DENSE)
    lanes8 = lax.iota(jnp.int32, L) * jnp.int32(CW)
    for c in range(NCH):
        sl = pl.ds(c * L, L)
        s = vals_cat_v[0, sl]
        for f in range(1, F_CAT):
            s = s + vals_cat_v[f, sl]
        for f in range(F_CROSS):
            gidx = (jnp.int32(c * (L * CW)) + lanes8
                    + (idx_cross_v[f, sl] & jnp.int32(CW - 1)))
            s = s + plsc.load_gather(cross8_v,
                                     [jnp.full((L,), f, jnp.int32), gidx])
        for f in range(F_DENSE):
            s = s + dense_v[f, sl] * wsplats[f]
        acc_v[sl] = s + bias_splat

    pltpu.sync_copy(acc_v, out.at[pl.ds(base, BPW)])


@jax.jit
def _poly2(catT, crossT, denseT, wcat, wcross, aux):
    mesh = plsc.VectorSubcoreMesh(core_axis_name="c", subcore_axis_name="s")
    return pl.kernel(
        _body,
        out_type=jax.ShapeDtypeStruct((B,), jnp.float32),
        mesh=mesh,
        scratch_types=[
            pltpu.VMEM((F_CAT, BPW), jnp.int32),
            pltpu.VMEM((F_CROSS, BPW), jnp.int32),
            pltpu.VMEM((F_CAT, BPW), jnp.float32),
            pltpu.VMEM((F_CROSS, BPW * CW), jnp.float32),
            pltpu.VMEM((F_DENSE, BPW), jnp.float32),
            pltpu.VMEM((BPW,), jnp.float32),
            pltpu.VMEM((L,), jnp.float32),
            pltpu.SemaphoreType.DMA,
            pltpu.SemaphoreType.DMA,
        ],
        compiler_params=pltpu.CompilerParams(needs_layout_passes=False),
    )(catT, crossT, denseT, wcat, wcross, aux)


def kernel(cat_idx, dense_x, cross_idx, W_cat, W_dense, W_cross, bias):
    catT = cat_idx.astype(jnp.int32).T
    crossT = cross_idx.astype(jnp.int32).T
    denseT = dense_x.T
    wcat = W_cat.reshape(-1)
    aux = jnp.concatenate(
        [W_dense.reshape(-1), bias.reshape(-1),
         jnp.zeros((L - F_DENSE - 1,), jnp.float32)])
    out = _poly2(catT, crossT, denseT, wcat, W_cross, aux)
    return out.reshape(B, 1)


# speedup vs baseline: 10.4858x; 1.0468x over previous
import jax
import jax.numpy as jnp
from jax import lax
from jax.experimental import pallas as pl
from jax.experimental.pallas import tpu as pltpu
from jax.experimental.pallas import tpu_sc as plsc

B = 4096
F_CAT = 26
V_CAT = 100000
F_DENSE = 13
F_CROSS = 6
V_CROSS = 1000000

NC = 2
NS = 16
L = 16
NW = NC * NS
BPW = B // NW
NCH = BPW // L
CW = 8


def _splat(aux_vec, i):
    return jnp.full((L,), aux_vec[i], jnp.float32)


def _body(catT, crossT, denseT, wcat, wcross, aux, out,
          idx_cat_v, idx_cross_v, vals_cat_v, cross8_v,
          dense_v, acc_v, aux_v, sem, csem):
    wid = lax.axis_index("s") * NC + lax.axis_index("c")
    base = wid * BPW

    stage = [
        pltpu.async_copy(catT.at[:, pl.ds(base, BPW)], idx_cat_v, sem),
        pltpu.async_copy(crossT.at[:, pl.ds(base, BPW)], idx_cross_v, sem),
        pltpu.async_copy(denseT.at[:, pl.ds(base, BPW)], dense_v, sem),
        pltpu.async_copy(aux, aux_v, sem),
    ]
    for cp in stage:
        cp.wait()

    for f in range(1, F_CAT):
        for c in range(NCH):
            sl = pl.ds(c * L, L)
            idx_cat_v[f, sl] = idx_cat_v[f, sl] + jnp.int32(f * V_CAT)
    copies = []
    for f in range(F_CAT):
        copies.append(
            pltpu.async_copy(wcat.at[idx_cat_v.at[f]], vals_cat_v.at[f], sem))

    for f in range(F_CROSS):
        def cross_chunk(c, carry, f=f):
            ivec = idx_cross_v[f, pl.ds(c * L, L)]
            col8 = ivec & jnp.int32(~(CW - 1))
            for l in range(L):
                off = pl.multiple_of(col8[l], CW)
                pltpu.async_copy(
                    wcross.at[f, pl.ds(off, CW)],
                    cross8_v.at[f, pl.ds(c * (L * CW) + l * CW, CW)],
                    csem)

            @pl.when(c > 2)
            def _drain_prev():
                pltpu.make_async_copy(
                    wcross.at[f, pl.ds(0, L * CW)],
                    cross8_v.at[f, pl.ds((c - 3) * (L * CW), L * CW)],
                    csem).wait()
            return carry
        lax.fori_loop(0, NCH, cross_chunk, 0)
        pltpu.make_async_copy(
            wcross.at[f, pl.ds(0, 3 * L * CW)],
            cross8_v.at[f, pl.ds((NCH - 3) * (L * CW), 3 * L * CW)],
            csem).wait()

    for cp in copies:
        cp.wait()

    aux_vec = aux_v[:]
    wsplats = [_splat(aux_vec, f) for f in range(F_DENSE)]
    bias_splat = _splat(aux_vec, F_DENSE)
    lanes8 = lax.iota(jnp.int32, L) * jnp.int32(CW)
    for c in range(NCH):
        sl = pl.ds(c * L, L)
        s = vals_cat_v[0, sl]
        for f in range(1, F_CAT):
            s = s + vals_cat_v[f, sl]
        for f in range(F_CROSS):
            gidx = (jnp.int32(c * (L * CW)) + lanes8
                    + (idx_cross_v[f, sl] & jnp.int32(CW - 1)))
            s = s + plsc.load_gather(cross8_v,
                                     [jnp.full((L,), f, jnp.int32), gidx])
        for f in range(F_DENSE):
            s = s + dense_v[f, sl] * wsplats[f]
        acc_v[sl] = s + bias_splat

    pltpu.sync_copy(acc_v, out.at[pl.ds(base, BPW)])


@jax.jit
def _poly2(catT, crossT, denseT, wcat, wcross, aux):
    mesh = plsc.VectorSubcoreMesh(core_axis_name="c", subcore_axis_name="s")
    return pl.kernel(
        _body,
        out_type=jax.ShapeDtypeStruct((B,), jnp.float32),
        mesh=mesh,
        scratch_types=[
            pltpu.VMEM((F_CAT, BPW), jnp.int32),
            pltpu.VMEM((F_CROSS, BPW), jnp.int32),
            pltpu.VMEM((F_CAT, BPW), jnp.float32),
            pltpu.VMEM((F_CROSS, BPW * CW), jnp.float32),
            pltpu.VMEM((F_DENSE, BPW), jnp.float32),
            pltpu.VMEM((BPW,), jnp.float32),
            pltpu.VMEM((L,), jnp.float32),
            pltpu.SemaphoreType.DMA,
            pltpu.SemaphoreType.DMA,
        ],
        compiler_params=pltpu.CompilerParams(needs_layout_passes=False),
    )(catT, crossT, denseT, wcat, wcross, aux)


def kernel(cat_idx, dense_x, cross_idx, W_cat, W_dense, W_cross, bias):
    catT = cat_idx.astype(jnp.int32).T
    crossT = cross_idx.astype(jnp.int32).T
    denseT = dense_x.T
    wcat = W_cat.reshape(-1)
    aux = jnp.concatenate(
        [W_dense.reshape(-1), bias.reshape(-1),
         jnp.zeros((L - F_DENSE - 1,), jnp.float32)])
    out = _poly2(catT, crossT, denseT, wcat, W_cross, aux)
    return out.reshape(B, 1)


# speedup vs baseline: 10.6356x; 1.0143x over previous
import jax
import jax.numpy as jnp
from jax import lax
from jax.experimental import pallas as pl
from jax.experimental.pallas import tpu as pltpu
from jax.experimental.pallas import tpu_sc as plsc

B = 4096
F_CAT = 26
V_CAT = 100000
F_DENSE = 13
F_CROSS = 6
V_CROSS = 1000000

NC = 2
NS = 16
L = 16
NW = NC * NS
BPW = B // NW
NCH = BPW // L
CW = 8


def _splat(aux_vec, i):
    return jnp.full((L,), aux_vec[i], jnp.float32)


def _body(catT, crossT, denseT, wcat, wcross, aux, out,
          idx_cat_v, idx_cross_v, vals_cat_v, cross8_v,
          dense_v, acc_v, aux_v, sem, csem):
    wid = lax.axis_index("s") * NC + lax.axis_index("c")
    base = wid * BPW

    stage = [
        pltpu.async_copy(catT.at[:, pl.ds(base, BPW)], idx_cat_v, sem),
        pltpu.async_copy(crossT.at[:, pl.ds(base, BPW)], idx_cross_v, sem),
        pltpu.async_copy(denseT.at[:, pl.ds(base, BPW)], dense_v, sem),
        pltpu.async_copy(aux, aux_v, sem),
    ]
    for cp in stage:
        cp.wait()

    for f in range(1, F_CAT):
        for c in range(NCH):
            sl = pl.ds(c * L, L)
            idx_cat_v[f, sl] = idx_cat_v[f, sl] + jnp.int32(f * V_CAT)
    copies = []
    for f in range(F_CAT):
        copies.append(
            pltpu.async_copy(wcat.at[idx_cat_v.at[f]], vals_cat_v.at[f], sem))

    for f in range(F_CROSS):
        def cross_chunk(c, carry, f=f):
            ivec = idx_cross_v[f, pl.ds(c * L, L)]
            col8 = ivec & jnp.int32(~(CW - 1))
            for l in range(L):
                off = pl.multiple_of(col8[l], CW)
                pltpu.async_copy(
                    wcross.at[f, pl.ds(off, CW)],
                    cross8_v.at[f, pl.ds(c * (L * CW) + l * CW, CW)],
                    csem)

            @pl.when(c > 3)
            def _drain_prev():
                pltpu.make_async_copy(
                    wcross.at[f, pl.ds(0, L * CW)],
                    cross8_v.at[f, pl.ds((c - 4) * (L * CW), L * CW)],
                    csem).wait()
            return carry
        lax.fori_loop(0, NCH, cross_chunk, 0)
        pltpu.make_async_copy(
            wcross.at[f, pl.ds(0, 4 * L * CW)],
            cross8_v.at[f, pl.ds((NCH - 4) * (L * CW), 4 * L * CW)],
            csem).wait()

    for cp in copies:
        cp.wait()

    aux_vec = aux_v[:]
    wsplats = [_splat(aux_vec, f) for f in range(F_DENSE)]
    bias_splat = _splat(aux_vec, F_DENSE)
    lanes8 = lax.iota(jnp.int32, L) * jnp.int32(CW)
    for c in range(NCH):
        sl = pl.ds(c * L, L)
        s = vals_cat_v[0, sl]
        for f in range(1, F_CAT):
            s = s + vals_cat_v[f, sl]
        for f in range(F_CROSS):
            gidx = (jnp.int32(c * (L * CW)) + lanes8
                    + (idx_cross_v[f, sl] & jnp.int32(CW - 1)))
            s = s + plsc.load_gather(cross8_v,
                                     [jnp.full((L,), f, jnp.int32), gidx])
        for f in range(F_DENSE):
            s = s + dense_v[f, sl] * wsplats[f]
        acc_v[sl] = s + bias_splat

    pltpu.sync_copy(acc_v, out.at[pl.ds(base, BPW)])


@jax.jit
def _poly2(catT, crossT, denseT, wcat, wcross, aux):
    mesh = plsc.VectorSubcoreMesh(core_axis_name="c", subcore_axis_name="s")
    return pl.kernel(
        _body,
        out_type=jax.ShapeDtypeStruct((B,), jnp.float32),
        mesh=mesh,
        scratch_types=[
            pltpu.VMEM((F_CAT, BPW), jnp.int32),
            pltpu.VMEM((F_CROSS, BPW), jnp.int32),
            pltpu.VMEM((F_CAT, BPW), jnp.float32),
            pltpu.VMEM((F_CROSS, BPW * CW), jnp.float32),
            pltpu.VMEM((F_DENSE, BPW), jnp.float32),
            pltpu.VMEM((BPW,), jnp.float32),
            pltpu.VMEM((L,), jnp.float32),
            pltpu.SemaphoreType.DMA,
            pltpu.SemaphoreType.DMA,
        ],
        compiler_params=pltpu.CompilerParams(needs_layout_passes=False),
    )(catT, crossT, denseT, wcat, wcross, aux)


def kernel(cat_idx, dense_x, cross_idx, W_cat, W_dense, W_cross, bias):
    catT = cat_idx.astype(jnp.int32).T
    crossT = cross_idx.astype(jnp.int32).T
    denseT = dense_x.T
    wcat = W_cat.reshape(-1)
    aux = jnp.concatenate(
        [W_dense.reshape(-1), bias.reshape(-1),
         jnp.zeros((L - F_DENSE - 1,), jnp.float32)])
    out = _poly2(catT, crossT, denseT, wcat, W_cross, aux)
    return out.reshape(B, 1)


# speedup vs baseline: 10.6458x; 1.0010x over previous
import jax
import jax.numpy as jnp
from jax import lax
from jax.experimental import pallas as pl
from jax.experimental.pallas import tpu as pltpu
from jax.experimental.pallas import tpu_sc as plsc

B = 4096
F_CAT = 26
V_CAT = 100000
F_DENSE = 13
F_CROSS = 6
V_CROSS = 1000000

NC = 2
NS = 16
L = 16
NW = NC * NS
BPW = B // NW
NCH = BPW // L
CW = 8


def _splat(aux_vec, i):
    return jnp.full((L,), aux_vec[i], jnp.float32)


def _body(catT, crossT, denseT, wcat, wcross, aux, out,
          idx_cat_v, idx_cross_v, vals_cat_v, cross8_v,
          dense_v, acc_v, aux_v, sem, csem):
    wid = lax.axis_index("s") * NC + lax.axis_index("c")
    base = wid * BPW

    stage = [
        pltpu.async_copy(catT.at[:, pl.ds(base, BPW)], idx_cat_v, sem),
        pltpu.async_copy(crossT.at[:, pl.ds(base, BPW)], idx_cross_v, sem),
        pltpu.async_copy(denseT.at[:, pl.ds(base, BPW)], dense_v, sem),
        pltpu.async_copy(aux, aux_v, sem),
    ]
    for cp in stage:
        cp.wait()

    for f in range(1, F_CAT):
        for c in range(NCH):
            sl = pl.ds(c * L, L)
            idx_cat_v[f, sl] = idx_cat_v[f, sl] + jnp.int32(f * V_CAT)
    copies = []
    for f in range(F_CAT):
        copies.append(
            pltpu.async_copy(wcat.at[idx_cat_v.at[f]], vals_cat_v.at[f], sem))

    for f in range(F_CROSS):
        def cross_chunk(c, carry, f=f):
            ivec = idx_cross_v[f, pl.ds(c * L, L)]
            col8 = ivec & jnp.int32(~(CW - 1))
            for l in range(L):
                off = pl.multiple_of(col8[l], CW)
                pltpu.async_copy(
                    wcross.at[f, pl.ds(off, CW)],
                    cross8_v.at[f, pl.ds(c * (L * CW) + l * CW, CW)],
                    csem)

            @pl.when(c > 3)
            def _drain_own():
                pltpu.make_async_copy(
                    wcross.at[f, pl.ds(0, L * CW)],
                    cross8_v.at[f, pl.ds((c - 4) * (L * CW), L * CW)],
                    csem).wait()

            if f > 0:
                @pl.when(c <= 3)
                def _drain_prev_field():
                    pltpu.make_async_copy(
                        wcross.at[f - 1, pl.ds(0, L * CW)],
                        cross8_v.at[f - 1, pl.ds((c + 4) * (L * CW), L * CW)],
                        csem).wait()
            return carry
        lax.fori_loop(0, NCH, cross_chunk, 0)
    pltpu.make_async_copy(
        wcross.at[F_CROSS - 1, pl.ds(0, 4 * L * CW)],
        cross8_v.at[F_CROSS - 1, pl.ds((NCH - 4) * (L * CW), 4 * L * CW)],
        csem).wait()

    for cp in copies:
        cp.wait()

    aux_vec = aux_v[:]
    wsplats = [_splat(aux_vec, f) for f in range(F_DENSE)]
    bias_splat = _splat(aux_vec, F_DENSE)
    lanes8 = lax.iota(jnp.int32, L) * jnp.int32(CW)
    for c in range(NCH):
        sl = pl.ds(c * L, L)
        s = vals_cat_v[0, sl]
        for f in range(1, F_CAT):
            s = s + vals_cat_v[f, sl]
        for f in range(F_CROSS):
            gidx = (jnp.int32(c * (L * CW)) + lanes8
                    + (idx_cross_v[f, sl] & jnp.int32(CW - 1)))
            s = s + plsc.load_gather(cross8_v,
                                     [jnp.full((L,), f, jnp.int32), gidx])
        for f in range(F_DENSE):
            s = s + dense_v[f, sl] * wsplats[f]
        acc_v[sl] = s + bias_splat

    pltpu.sync_copy(acc_v, out.at[pl.ds(base, BPW)])


@jax.jit
def _poly2(catT, crossT, denseT, wcat, wcross, aux):
    mesh = plsc.VectorSubcoreMesh(core_axis_name="c", subcore_axis_name="s")
    return pl.kernel(
        _body,
        out_type=jax.ShapeDtypeStruct((B,), jnp.float32),
        mesh=mesh,
        scratch_types=[
            pltpu.VMEM((F_CAT, BPW), jnp.int32),
            pltpu.VMEM((F_CROSS, BPW), jnp.int32),
            pltpu.VMEM((F_CAT, BPW), jnp.float32),
            pltpu.VMEM((F_CROSS, BPW * CW), jnp.float32),
            pltpu.VMEM((F_DENSE, BPW), jnp.float32),
            pltpu.VMEM((BPW,), jnp.float32),
            pltpu.VMEM((L,), jnp.float32),
            pltpu.SemaphoreType.DMA,
            pltpu.SemaphoreType.DMA,
        ],
        compiler_params=pltpu.CompilerParams(needs_layout_passes=False),
    )(catT, crossT, denseT, wcat, wcross, aux)


def kernel(cat_idx, dense_x, cross_idx, W_cat, W_dense, W_cross, bias):
    catT = cat_idx.astype(jnp.int32).T
    crossT = cross_idx.astype(jnp.int32).T
    denseT = dense_x.T
    wcat = W_cat.reshape(-1)
    aux = jnp.concatenate(
        [W_dense.reshape(-1), bias.reshape(-1),
         jnp.zeros((L - F_DENSE - 1,), jnp.float32)])
    out = _poly2(catT, crossT, denseT, wcat, W_cross, aux)
    return out.reshape(B, 1)
